# baseline (speedup 1.0000x reference)
import jax
import jax.numpy as jnp
from jax import lax
from jax.experimental import pallas as pl
from jax.experimental.pallas import tpu as pltpu
from jax.experimental.pallas import tpu_sc as plsc

D_MODEL = 32
NC, NS = 2, 16
NW = NC * NS
SBLK = 128


def _embed_body(idx_hbm, tab_hbm, out_hbm, idx_v, rows_v, trans_v,
                gsem0, gsem1, isem0, isem1, osem0, osem1):
    wid = lax.axis_index("s") * NC + lax.axis_index("c")
    c0 = wid * SBLK
    nt = idx_hbm.shape[0]
    gsems = (gsem0, gsem1)
    isems = (isem0, isem1)
    osems = (osem0, osem1)
    iota16 = lax.iota(jnp.int32, 16)

    def stage_idx(t, slot, sem):
        pltpu.async_copy(idx_hbm.at[t, pl.ds(c0, SBLK)], idx_v.at[slot], sem)

    def wait_idx(slot):
        pltpu.make_async_copy(idx_hbm.at[0, pl.ds(c0, SBLK)],
                              idx_v.at[slot], isems[slot]).wait()

    def fire_gather(slot):
        pltpu.async_copy(tab_hbm.at[idx_v.at[slot]], rows_v.at[slot],
                         gsems[slot])

    def wait_gather(slot):
        pltpu.make_async_copy(tab_hbm.at[idx_v.at[slot]], rows_v.at[slot],
                              gsems[slot]).wait()

    def fire_writes(t, slot):
        for r in range(4):
            pltpu.async_copy(trans_v.at[slot, r], out_hbm.at[t, r, wid],
                             osems[slot])

    def wait_writes(t, slot):
        for r in range(4):
            pltpu.make_async_copy(trans_v.at[slot, r],
                                  out_hbm.at[t, r, wid], osems[slot]).wait()

    def transpose(slot):
        for r in range(4):
            for sl in range(8):
                col = jnp.full((16,), 8 * r + sl, jnp.int32)
                slotv = jnp.full((16,), slot, jnp.int32)
                for lg in range(8):
                    vec = plsc.load_gather(rows_v,
                                           [slotv, iota16 + 16 * lg, col])
                    trans_v[slot, r, sl, pl.ds(16 * lg, 16)] = vec

    pltpu.sync_copy(idx_hbm.at[0, pl.ds(c0, SBLK)], idx_v.at[0])
    stage_idx(1, 1, isem1)
    fire_gather(0)
    fire_writes(0, 0)
    fire_writes(1, 1)

    @pl.loop(0, (nt - 2) // 2)
    def pair_body(i):
        for sub in (0, 1):
            t = 2 * i + sub
            slot = sub
            wait_gather(slot)
            stage_idx(t + 2, slot, isems[slot])
            wait_idx(1 - slot)
            fire_gather(1 - slot)
            wait_writes(t, slot)
            transpose(slot)
            fire_writes(t, slot)

    t = nt - 2
    wait_gather(0)
    wait_idx(1)
    fire_gather(1)
    wait_writes(t, 0)
    transpose(0)
    fire_writes(t, 0)
    wait_gather(1)
    wait_writes(t + 1, 1)
    transpose(1)
    fire_writes(t + 1, 1)
    wait_writes(t, 0)
    wait_writes(t + 1, 1)


def kernel(tokens, weights):
    nseq, seq_len = tokens.shape
    assert nseq % (SBLK * NW) == 0 or nseq == SBLK * NW
    assert seq_len % 2 == 0

    mesh = plsc.VectorSubcoreMesh(core_axis_name="c", subcore_axis_name="s")
    grid_fn = pl.kernel(
        _embed_body,
        out_type=jax.ShapeDtypeStruct((seq_len, 4, nseq // SBLK, 8, SBLK),
                                      jnp.float32),
        mesh=mesh,
        scratch_types=[
            pltpu.VMEM((2, SBLK), jnp.int32),
            pltpu.VMEM((2, SBLK, D_MODEL), jnp.float32),
            pltpu.VMEM((2, 4, 8, SBLK), jnp.float32),
            pltpu.SemaphoreType.DMA,
            pltpu.SemaphoreType.DMA,
            pltpu.SemaphoreType.DMA,
            pltpu.SemaphoreType.DMA,
            pltpu.SemaphoreType.DMA,
            pltpu.SemaphoreType.DMA,
        ],
        compiler_params=pltpu.CompilerParams(use_tc_tiling_on_sc=False,
                                             needs_layout_passes=False),
    )
    tokens_t = tokens.T.astype(jnp.int32)
    res = grid_fn(tokens_t, weights)
    return res.transpose(2, 4, 0, 1, 3).reshape(nseq, seq_len, D_MODEL)


# speedup vs baseline: 1.1425x; 1.1425x over previous
import functools

import jax
import jax.numpy as jnp
from jax import lax
from jax.experimental import pallas as pl
from jax.experimental.pallas import tpu as pltpu
from jax.experimental.pallas import tpu_sc as plsc

D_MODEL = 32
NC, NS = 2, 16
NW = NC * NS
SEQ_CHUNK = 2
STREAM = 40


def _embed_body(idx_hbm, tab_hbm, out_hbm, idx_v, rows_v, gsem0, gsem1, osem,
                *, seq_len, seqs_per_w, nchunk):
    wid = lax.axis_index("s") * NC + lax.axis_index("c")
    s0 = wid * seqs_per_w
    gsems = (gsem0, gsem1)
    nstream = SEQ_CHUNK * seq_len // STREAM
    per_seq = seq_len // STREAM

    def stream_slices(slot, j):
        r, c = j // per_seq, (j % per_seq) * STREAM
        return (idx_v.at[slot, r, pl.ds(c, STREAM)],
                rows_v.at[slot, r, pl.ds(c, STREAM)])

    def load_and_fire(g, slot):
        pltpu.sync_copy(idx_hbm.at[pl.ds(s0 + g * SEQ_CHUNK, SEQ_CHUNK)],
                        idx_v.at[slot])
        for j in range(nstream):
            isl, rsl = stream_slices(slot, j)
            pltpu.async_copy(tab_hbm.at[isl], rsl, gsems[slot])

    def drain_gathers(slot):
        for j in range(nstream):
            isl, rsl = stream_slices(slot, j)
            pltpu.make_async_copy(tab_hbm.at[isl], rsl, gsems[slot]).wait()

    def write_out(g, slot):
        dst = out_hbm.at[pl.ds(s0 + g * SEQ_CHUNK, SEQ_CHUNK)]
        pltpu.async_copy(rows_v.at[slot], dst, osem)
        pltpu.make_async_copy(rows_v.at[slot], dst, osem).wait()

    load_and_fire(0, 0)

    npairs = (nchunk - 1) // 2

    @pl.loop(0, npairs)
    def pair_body(i):
        for sub in (0, 1):
            g = 2 * i + sub
            load_and_fire(g + 1, 1 - sub)
            drain_gathers(sub)
            write_out(g, sub)

    if (nchunk - 1) % 2 == 1:
        g = nchunk - 2
        load_and_fire(g + 1, (g + 1) % 2)
        drain_gathers(g % 2)
        write_out(g, g % 2)

    g = nchunk - 1
    drain_gathers(g % 2)
    write_out(g, g % 2)


def kernel(tokens, weights):
    nseq, seq_len = tokens.shape
    assert nseq % NW == 0 and seq_len % STREAM == 0
    seqs_per_w = nseq // NW
    assert seqs_per_w % SEQ_CHUNK == 0
    nchunk = seqs_per_w // SEQ_CHUNK

    mesh = plsc.VectorSubcoreMesh(core_axis_name="c", subcore_axis_name="s")
    grid_fn = pl.kernel(
        functools.partial(_embed_body, seq_len=seq_len,
                          seqs_per_w=seqs_per_w, nchunk=nchunk),
        out_type=jax.ShapeDtypeStruct((nseq, seq_len, D_MODEL), jnp.float32),
        mesh=mesh,
        scratch_types=[
            pltpu.VMEM((2, SEQ_CHUNK, seq_len), jnp.int32),
            pltpu.VMEM((2, SEQ_CHUNK, seq_len, D_MODEL), jnp.float32),
            pltpu.SemaphoreType.DMA,
            pltpu.SemaphoreType.DMA,
            pltpu.SemaphoreType.DMA,
        ],
        compiler_params=pltpu.CompilerParams(use_tc_tiling_on_sc=False),
    )
    return grid_fn(tokens.astype(jnp.int32), weights)


# speedup vs baseline: 1.1495x; 1.0061x over previous
import functools

import jax
import jax.numpy as jnp
from jax import lax
from jax.experimental import pallas as pl
from jax.experimental.pallas import tpu as pltpu
from jax.experimental.pallas import tpu_sc as plsc

D_MODEL = 32
NC, NS = 2, 16
NW = NC * NS
CHUNK = 512
STREAM = 128


def _embed_body(idx_hbm, tab_hbm, out_hbm, idx_v, rows_v, gsem0, gsem1, osem,
                *, nchunk):
    wid = lax.axis_index("s") * NC + lax.axis_index("c")
    i0 = wid * nchunk * CHUNK
    gsems = (gsem0, gsem1)
    nstream = CHUNK // STREAM

    def load_and_fire(g, slot):
        pltpu.sync_copy(idx_hbm.at[pl.ds(i0 + g * CHUNK, CHUNK)],
                        idx_v.at[slot])
        for j in range(nstream):
            isl = idx_v.at[slot, pl.ds(j * STREAM, STREAM)]
            rsl = rows_v.at[slot, pl.ds(j * STREAM, STREAM)]
            pltpu.async_copy(tab_hbm.at[isl], rsl, gsems[slot])

    def drain_gathers(slot):
        for j in range(nstream):
            isl = idx_v.at[slot, pl.ds(j * STREAM, STREAM)]
            rsl = rows_v.at[slot, pl.ds(j * STREAM, STREAM)]
            pltpu.make_async_copy(tab_hbm.at[isl], rsl, gsems[slot]).wait()

    def write_out(g, slot):
        dst = out_hbm.at[pl.ds(i0 + g * CHUNK, CHUNK)]
        pltpu.async_copy(rows_v.at[slot], dst, osem)
        pltpu.make_async_copy(rows_v.at[slot], dst, osem).wait()

    load_and_fire(0, 0)

    npairs = (nchunk - 1) // 2

    @pl.loop(0, npairs)
    def pair_body(i):
        for sub in (0, 1):
            g = 2 * i + sub
            load_and_fire(g + 1, 1 - sub)
            drain_gathers(sub)
            write_out(g, sub)

    if (nchunk - 1) % 2 == 1:
        g = nchunk - 2
        load_and_fire(g + 1, (g + 1) % 2)
        drain_gathers(g % 2)
        write_out(g, g % 2)

    g = nchunk - 1
    drain_gathers(g % 2)
    write_out(g, g % 2)


def kernel(tokens, weights):
    nseq, seq_len = tokens.shape
    nflat = nseq * seq_len
    assert nflat % (NW * CHUNK) == 0
    nchunk = nflat // (NW * CHUNK)

    mesh = plsc.VectorSubcoreMesh(core_axis_name="c", subcore_axis_name="s")
    grid_fn = pl.kernel(
        functools.partial(_embed_body, nchunk=nchunk),
        out_type=jax.ShapeDtypeStruct((nflat, D_MODEL), jnp.float32),
        mesh=mesh,
        scratch_types=[
            pltpu.VMEM((2, CHUNK), jnp.int32),
            pltpu.VMEM((2, CHUNK, D_MODEL), jnp.float32),
            pltpu.SemaphoreType.DMA,
            pltpu.SemaphoreType.DMA,
            pltpu.SemaphoreType.DMA,
        ],
        compiler_params=pltpu.CompilerParams(use_tc_tiling_on_sc=False),
    )
    res = grid_fn(tokens.reshape(nflat).astype(jnp.int32), weights)
    return res.reshape(nseq, seq_len, D_MODEL)


# speedup vs baseline: 1.1627x; 1.0115x over previous
import functools

import jax
import jax.numpy as jnp
from jax import lax
from jax.experimental import pallas as pl
from jax.experimental.pallas import tpu as pltpu
from jax.experimental.pallas import tpu_sc as plsc

D_MODEL = 32
NC, NS = 2, 16
NW = NC * NS
CHUNK = 1024
STREAM = 128


def _embed_body(idx_hbm, tab_hbm, out_hbm, idx_v, rows_v, gsem0, gsem1, osem,
                *, nchunk):
    wid = lax.axis_index("s") * NC + lax.axis_index("c")
    i0 = wid * nchunk * CHUNK
    gsems = (gsem0, gsem1)
    nstream = CHUNK // STREAM

    def load_and_fire(g, slot):
        pltpu.sync_copy(idx_hbm.at[pl.ds(i0 + g * CHUNK, CHUNK)],
                        idx_v.at[slot])
        for j in range(nstream):
            isl = idx_v.at[slot, pl.ds(j * STREAM, STREAM)]
            rsl = rows_v.at[slot, pl.ds(j * STREAM, STREAM)]
            pltpu.async_copy(tab_hbm.at[isl], rsl, gsems[slot])

    def drain_gathers(slot):
        for j in range(nstream):
            isl = idx_v.at[slot, pl.ds(j * STREAM, STREAM)]
            rsl = rows_v.at[slot, pl.ds(j * STREAM, STREAM)]
            pltpu.make_async_copy(tab_hbm.at[isl], rsl, gsems[slot]).wait()

    def write_out(g, slot):
        dst = out_hbm.at[pl.ds(i0 + g * CHUNK, CHUNK)]
        pltpu.async_copy(rows_v.at[slot], dst, osem)
        pltpu.make_async_copy(rows_v.at[slot], dst, osem).wait()

    load_and_fire(0, 0)

    npairs = (nchunk - 1) // 2

    @pl.loop(0, npairs)
    def pair_body(i):
        for sub in (0, 1):
            g = 2 * i + sub
            load_and_fire(g + 1, 1 - sub)
            drain_gathers(sub)
            write_out(g, sub)

    if (nchunk - 1) % 2 == 1:
        g = nchunk - 2
        load_and_fire(g + 1, (g + 1) % 2)
        drain_gathers(g % 2)
        write_out(g, g % 2)

    g = nchunk - 1
    drain_gathers(g % 2)
    write_out(g, g % 2)


def kernel(tokens, weights):
    nseq, seq_len = tokens.shape
    nflat = nseq * seq_len
    assert nflat % (NW * CHUNK) == 0
    nchunk = nflat // (NW * CHUNK)

    mesh = plsc.VectorSubcoreMesh(core_axis_name="c", subcore_axis_name="s")
    grid_fn = pl.kernel(
        functools.partial(_embed_body, nchunk=nchunk),
        out_type=jax.ShapeDtypeStruct((nflat, D_MODEL), jnp.float32),
        mesh=mesh,
        scratch_types=[
            pltpu.VMEM((2, CHUNK), jnp.int32),
            pltpu.VMEM((2, CHUNK, D_MODEL), jnp.float32),
            pltpu.SemaphoreType.DMA,
            pltpu.SemaphoreType.DMA,
            pltpu.SemaphoreType.DMA,
        ],
        compiler_params=pltpu.CompilerParams(use_tc_tiling_on_sc=False),
    )
    res = grid_fn(tokens.reshape(nflat).astype(jnp.int32), weights)
    return res.reshape(nseq, seq_len, D_MODEL)
